# Initial kernel scaffold; baseline (speedup 1.0000x reference)
import jax
import jax.numpy as jnp
from jax.experimental import pallas as pl


def kernel(x, text_feature, Wg_x, Wg_t, bg, W1, b1, W2, b2):
    raise NotImplementedError("write your pallas kernel here")


# speedup vs baseline: 1.3010x; 1.3010x over previous
import jax
import jax.numpy as jnp
from jax.experimental import pallas as pl
from jax.experimental.pallas import tpu as pltpu

_B, _C, _H, _W = 4, 192, 24, 24
_DT, _E, _K = 512, 8, 2
_HW = _H * _W


def _gate_kernel(x_ref, t_ref, wx_ref, wt_ref, bg_ref, idx_ref, g_ref, aux_ref):
    xp = jnp.mean(x_ref[...], axis=1)
    logits = (
        jnp.dot(xp, wx_ref[...], preferred_element_type=jnp.float32)
        + jnp.dot(t_ref[...], wt_ref[...], preferred_element_type=jnp.float32)
        + bg_ref[...]
    )
    ii = jax.lax.broadcasted_iota(jnp.int32, (_B, _E), 1)
    m1 = jnp.max(logits, axis=1, keepdims=True)
    i1 = jnp.min(jnp.where(logits == m1, ii, _E), axis=1, keepdims=True)
    masked = jnp.where(ii == i1, -jnp.inf, logits)
    m2 = jnp.max(masked, axis=1, keepdims=True)
    i2 = jnp.min(jnp.where(masked == m2, ii, _E), axis=1, keepdims=True)
    g1 = 1.0 / (1.0 + jnp.exp(m2 - m1))
    g2 = 1.0 - g1
    sel1 = ii == i1
    sel2 = ii == i2
    gates = jnp.where(sel1, g1, 0.0) + jnp.where(sel2, g2, 0.0)
    importance = jnp.sum(gates, axis=0, keepdims=True)
    load = jnp.sum((sel1 | sel2).astype(jnp.float32), axis=0, keepdims=True)

    def _cv(v):
        m = jnp.mean(v)
        var = jnp.mean(v * v) - m * m
        return var / (m * m + 1e-10)

    aux_ref[...] = (_cv(importance) + _cv(load)).reshape(1, 1)
    idx_ref[...] = jnp.concatenate([i1, i2], axis=1)
    g_ref[...] = jnp.concatenate([g1, g2], axis=1)


def _conv3x3(x2d, h, w, w_ref, bvec):
    x3 = x2d.reshape(h, w, _C)
    xp = jnp.pad(x3, ((1, 1), (1, 1), (0, 0)))
    acc = jnp.zeros((h * w, _C), jnp.float32)
    for dh in range(3):
        for dw in range(3):
            xs = xp[dh : dh + h, dw : dw + w, :].reshape(h * w, _C)
            acc = acc + jnp.dot(
                xs, w_ref[0, dh * 3 + dw], preferred_element_type=jnp.float32
            )
    return acc + bvec[None, :]


def _expert_block(x2d, h, w, w1_ref, b1_ref, w2_ref, b2_ref):
    y = _conv3x3(x2d, h, w, w1_ref, b1_ref[0, 0])
    y = jax.nn.gelu(y)
    return _conv3x3(y, h, w, w2_ref, b2_ref[0, 0])


def _expert_kernel(idx_ref, x_ref, w1_ref, b1_ref, w2_ref, b2_ref, g_ref, out_ref):
    s = pl.program_id(0)
    e = idx_ref[s]
    cls = jax.lax.rem(e, 3)
    g = g_ref[s, 0]
    x2d = x_ref[0]

    def _accum(y2d):
        contrib = g * y2d

        @pl.when(s % _K == 0)
        def _():
            out_ref[0] = contrib

        @pl.when(s % _K != 0)
        def _():
            out_ref[0] = out_ref[0] + contrib

    @pl.when(cls == 0)
    def _():
        _accum(_expert_block(x2d, _H, _W, w1_ref, b1_ref, w2_ref, b2_ref))

    @pl.when(cls == 1)
    def _():
        x3 = x2d.reshape(_H, _W, _C)
        xup = jnp.repeat(jnp.repeat(x3, 2, axis=0), 2, axis=1)
        y = _expert_block(
            xup.reshape(4 * _HW, _C), 2 * _H, 2 * _W, w1_ref, b1_ref, w2_ref, b2_ref
        )
        y4 = y.reshape(_H, 2, _W, 2, _C)
        _accum(jnp.max(jnp.max(y4, axis=3), axis=1).reshape(_HW, _C))

    @pl.when(cls == 2)
    def _():
        x4 = x2d.reshape(_H // 2, 2, _W // 2, 2, _C)
        xdn = jnp.max(jnp.max(x4, axis=3), axis=1)
        y = _expert_block(
            xdn.reshape(_HW // 4, _C), _H // 2, _W // 2, w1_ref, b1_ref, w2_ref, b2_ref
        )
        y3 = y.reshape(_H // 2, _W // 2, _C)
        yup = jnp.repeat(jnp.repeat(y3, 2, axis=0), 2, axis=1)
        _accum(yup.reshape(_HW, _C))


def kernel(x, text_feature, Wg_x, Wg_t, bg, W1, b1, W2, b2):
    x_hwc = x.transpose(0, 2, 3, 1).reshape(_B, _HW, _C)

    idx, topg, aux = pl.pallas_call(
        _gate_kernel,
        out_shape=[
            jax.ShapeDtypeStruct((_B, _K), jnp.int32),
            jax.ShapeDtypeStruct((_B, _K), jnp.float32),
            jax.ShapeDtypeStruct((1, 1), jnp.float32),
        ],
    )(x_hwc, text_feature, Wg_x, Wg_t, bg.reshape(1, _E))

    idx_flat = idx.reshape(_B * _K)
    g_flat = topg.reshape(_B * _K, 1)

    W1t = W1.transpose(0, 3, 4, 2, 1).reshape(_E, 9, _C, _C)
    W2t = W2.transpose(0, 3, 4, 2, 1).reshape(_E, 9, _C, _C)

    grid_spec = pltpu.PrefetchScalarGridSpec(
        num_scalar_prefetch=1,
        grid=(_B * _K,),
        in_specs=[
            pl.BlockSpec((1, _HW, _C), lambda s, idx: (s // _K, 0, 0)),
            pl.BlockSpec((1, 9, _C, _C), lambda s, idx: (idx[s], 0, 0, 0)),
            pl.BlockSpec((1, 1, _C), lambda s, idx: (idx[s], 0, 0)),
            pl.BlockSpec((1, 9, _C, _C), lambda s, idx: (idx[s], 0, 0, 0)),
            pl.BlockSpec((1, 1, _C), lambda s, idx: (idx[s], 0, 0)),
            pl.BlockSpec((_B * _K, 1), lambda s, idx: (0, 0)),
        ],
        out_specs=pl.BlockSpec((1, _HW, _C), lambda s, idx: (s // _K, 0, 0)),
    )
    out_hwc = pl.pallas_call(
        _expert_kernel,
        grid_spec=grid_spec,
        out_shape=jax.ShapeDtypeStruct((_B, _HW, _C), jnp.float32),
    )(idx_flat, x_hwc, W1t, b1.reshape(_E, 1, _C), W2t, b2.reshape(_E, 1, _C), g_flat)

    out = out_hwc.reshape(_B, _H, _W, _C).transpose(0, 3, 1, 2)
    return out, aux.reshape(())


# speedup vs baseline: 1.3777x; 1.0590x over previous
import jax
import jax.numpy as jnp
from jax.experimental import pallas as pl
from jax.experimental.pallas import tpu as pltpu

_B, _C, _H, _W = 4, 192, 24, 24
_DT, _E, _K = 512, 8, 2
_HW = _H * _W


def _gate_kernel(x_ref, t_ref, wx_ref, wt_ref, bg_ref, idx_ref, g_ref, aux_ref):
    xp = jnp.mean(x_ref[...], axis=2)
    logits = (
        jnp.dot(xp, wx_ref[...], preferred_element_type=jnp.float32)
        + jnp.dot(t_ref[...], wt_ref[...], preferred_element_type=jnp.float32)
        + bg_ref[...]
    )
    ii = jax.lax.broadcasted_iota(jnp.int32, (_B, _E), 1)
    m1 = jnp.max(logits, axis=1, keepdims=True)
    i1 = jnp.min(jnp.where(logits == m1, ii, _E), axis=1, keepdims=True)
    masked = jnp.where(ii == i1, -jnp.inf, logits)
    m2 = jnp.max(masked, axis=1, keepdims=True)
    i2 = jnp.min(jnp.where(masked == m2, ii, _E), axis=1, keepdims=True)
    g1 = 1.0 / (1.0 + jnp.exp(m2 - m1))
    g2 = 1.0 - g1
    sel1 = ii == i1
    sel2 = ii == i2
    gates = jnp.where(sel1, g1, 0.0) + jnp.where(sel2, g2, 0.0)
    importance = jnp.sum(gates, axis=0, keepdims=True)
    load = jnp.sum((sel1 | sel2).astype(jnp.float32), axis=0, keepdims=True)

    def _cv(v):
        m = jnp.mean(v)
        var = jnp.mean(v * v) - m * m
        return var / (m * m + 1e-10)

    aux_ref[...] = (_cv(importance) + _cv(load)).reshape(1, 1)
    idx_ref[...] = jnp.concatenate([i1, i2], axis=1)
    g_ref[...] = jnp.concatenate([g1, g2], axis=1)


def _conv3x3(xT, h, w, wn, bvec):
    hw = h * w
    p = jax.lax.broadcasted_iota(jnp.int32, (1, hw), 1)
    ph = p // w
    pw = jax.lax.rem(p, w)
    cols = []
    for dh in (-1, 0, 1):
        for dw in (-1, 0, 1):
            s = dh * w + dw
            xs = xT if s == 0 else jnp.roll(xT, -s, axis=1)
            m = (ph + dh >= 0) & (ph + dh < h) & (pw + dw >= 0) & (pw + dw < w)
            cols.append(jnp.where(m, xs, 0.0))
    xcol = jnp.concatenate(cols, axis=0)
    y = jax.lax.dot_general(
        wn, xcol, (((1,), (0,)), ((), ())), preferred_element_type=jnp.float32
    )
    return y + bvec


def _expert_block(xT, h, w, w1_ref, b1_ref, w2_ref, b2_ref):
    y = _conv3x3(xT, h, w, w1_ref[0], b1_ref[0])
    y = jax.nn.gelu(y)
    return _conv3x3(y, h, w, w2_ref[0], b2_ref[0])


def _upsample_mat(h, w):
    q = jax.lax.broadcasted_iota(jnp.int32, (h * w, 4 * h * w), 1)
    p = jax.lax.broadcasted_iota(jnp.int32, (h * w, 4 * h * w), 0)
    qh = q // (2 * w)
    qw = jax.lax.rem(q, 2 * w)
    src = (qh // 2) * w + qw // 2
    return (p == src).astype(jnp.float32)


def _pool_select_mat(h, w):
    q = jax.lax.broadcasted_iota(jnp.int32, (4 * h * w, h * w), 0)
    p = jax.lax.broadcasted_iota(jnp.int32, (4 * h * w, h * w), 1)
    ph = p // w
    pw = jax.lax.rem(p, w)
    src = (2 * ph) * (2 * w) + 2 * pw
    return (q == src).astype(jnp.float32)


def _pool_max2(y, w2):
    m1 = jnp.maximum(y, jnp.roll(y, -1, axis=1))
    return jnp.maximum(m1, jnp.roll(m1, -w2, axis=1))


def _mm(a, b):
    return jax.lax.dot_general(
        a, b, (((1,), (0,)), ((), ())), preferred_element_type=jnp.float32
    )


def _expert_kernel(idx_ref, x_ref, w1_ref, b1_ref, w2_ref, b2_ref, g_ref, out_ref):
    s = pl.program_id(0)
    e = idx_ref[s]
    cls = jax.lax.rem(e, 3)
    g = g_ref[s, 0]
    xT = x_ref[0]

    def _accum(y):
        contrib = g * y

        @pl.when(s % _K == 0)
        def _():
            out_ref[0] = contrib

        @pl.when(s % _K != 0)
        def _():
            out_ref[0] = out_ref[0] + contrib

    @pl.when(cls == 0)
    def _():
        _accum(_expert_block(xT, _H, _W, w1_ref, b1_ref, w2_ref, b2_ref))

    @pl.when(cls == 1)
    def _():
        xup = _mm(xT, _upsample_mat(_H, _W))
        y = _expert_block(xup, 2 * _H, 2 * _W, w1_ref, b1_ref, w2_ref, b2_ref)
        _accum(_mm(_pool_max2(y, 2 * _W), _pool_select_mat(_H, _W)))

    @pl.when(cls == 2)
    def _():
        xdn = _mm(_pool_max2(xT, _W), _pool_select_mat(_H // 2, _W // 2))
        y = _expert_block(xdn, _H // 2, _W // 2, w1_ref, b1_ref, w2_ref, b2_ref)
        _accum(_mm(y, _upsample_mat(_H // 2, _W // 2)))


def kernel(x, text_feature, Wg_x, Wg_t, bg, W1, b1, W2, b2):
    x_chw = x.reshape(_B, _C, _HW)

    idx, topg, aux = pl.pallas_call(
        _gate_kernel,
        out_shape=[
            jax.ShapeDtypeStruct((_B, _K), jnp.int32),
            jax.ShapeDtypeStruct((_B, _K), jnp.float32),
            jax.ShapeDtypeStruct((1, 1), jnp.float32),
        ],
    )(x_chw, text_feature, Wg_x, Wg_t, bg.reshape(1, _E))

    idx_flat = idx.reshape(_B * _K)
    g_flat = topg.reshape(_B * _K, 1)

    grid_spec = pltpu.PrefetchScalarGridSpec(
        num_scalar_prefetch=1,
        grid=(_B * _K,),
        in_specs=[
            pl.BlockSpec((1, _C, _HW), lambda s, idx: (s // _K, 0, 0)),
            pl.BlockSpec((1, _C, 9 * _C), lambda s, idx: (idx[s], 0, 0)),
            pl.BlockSpec((1, _C, 1), lambda s, idx: (idx[s], 0, 0)),
            pl.BlockSpec((1, _C, 9 * _C), lambda s, idx: (idx[s], 0, 0)),
            pl.BlockSpec((1, _C, 1), lambda s, idx: (idx[s], 0, 0)),
            pl.BlockSpec((_B * _K, 1), lambda s, idx: (0, 0)),
        ],
        out_specs=pl.BlockSpec((1, _C, _HW), lambda s, idx: (s // _K, 0, 0)),
    )
    out_flat = pl.pallas_call(
        _expert_kernel,
        grid_spec=grid_spec,
        out_shape=jax.ShapeDtypeStruct((_B, _C, _HW), jnp.float32),
    )(
        idx_flat,
        x_chw,
        jnp.swapaxes(W1.reshape(_E, _C, _C, 9), 2, 3).reshape(_E, _C, 9 * _C),
        b1.reshape(_E, _C, 1),
        jnp.swapaxes(W2.reshape(_E, _C, _C, 9), 2, 3).reshape(_E, _C, 9 * _C),
        b2.reshape(_E, _C, 1),
        g_flat,
    )

    return out_flat.reshape(_B, _C, _H, _W), aux.reshape(())


# speedup vs baseline: 2.2977x; 1.6678x over previous
import jax
import jax.numpy as jnp
from jax.experimental import pallas as pl
from jax.experimental.pallas import tpu as pltpu

_B, _C, _H, _W = 4, 192, 24, 24
_DT, _E, _K = 512, 8, 2
_HW = _H * _W


def _gate_kernel(x_ref, t_ref, wx_ref, wt_ref, bg_ref, idx_ref, g_ref, aux_ref):
    xp = jnp.mean(x_ref[...], axis=2)
    logits = (
        jnp.dot(xp, wx_ref[...], preferred_element_type=jnp.float32)
        + jnp.dot(t_ref[...], wt_ref[...], preferred_element_type=jnp.float32)
        + bg_ref[...]
    )
    ii = jax.lax.broadcasted_iota(jnp.int32, (_B, _E), 1)
    m1 = jnp.max(logits, axis=1, keepdims=True)
    i1 = jnp.min(jnp.where(logits == m1, ii, _E), axis=1, keepdims=True)
    masked = jnp.where(ii == i1, -jnp.inf, logits)
    m2 = jnp.max(masked, axis=1, keepdims=True)
    i2 = jnp.min(jnp.where(masked == m2, ii, _E), axis=1, keepdims=True)
    g1 = 1.0 / (1.0 + jnp.exp(m2 - m1))
    g2 = 1.0 - g1
    sel1 = ii == i1
    sel2 = ii == i2
    gates = jnp.where(sel1, g1, 0.0) + jnp.where(sel2, g2, 0.0)
    importance = jnp.sum(gates, axis=0, keepdims=True)
    load = jnp.sum((sel1 | sel2).astype(jnp.float32), axis=0, keepdims=True)

    def _cv(v):
        m = jnp.mean(v)
        var = jnp.mean(v * v) - m * m
        return var / (m * m + 1e-10)

    aux_ref[...] = (_cv(importance) + _cv(load)).reshape(1, 1)
    idx_ref[...] = jnp.concatenate([i1, i2], axis=1)
    g_ref[...] = jnp.concatenate([g1, g2], axis=1)


def _conv3x3(xT, h, w, wn, bvec):
    hw = h * w
    p = jax.lax.broadcasted_iota(jnp.int32, (1, hw), 1)
    ph = p // w
    pw = jax.lax.rem(p, w)
    cols = []
    for dh in (-1, 0, 1):
        for dw in (-1, 0, 1):
            s = dh * w + dw
            xs = xT if s == 0 else jnp.roll(xT, -s, axis=1)
            m = (ph + dh >= 0) & (ph + dh < h) & (pw + dw >= 0) & (pw + dw < w)
            cols.append(jnp.where(m, xs, 0.0))
    xcol = jnp.concatenate(cols, axis=0)
    y = jax.lax.dot_general(
        wn, xcol, (((1,), (0,)), ((), ())), preferred_element_type=jnp.float32
    )
    return y + bvec


def _reorder_w(wn, perm):
    return jax.lax.dot_general(
        wn.astype(jnp.bfloat16),
        perm,
        (((1,), (0,)), ((), ())),
        preferred_element_type=jnp.float32,
    )


def _expert_block(xT, h, w, w1_ref, b1_ref, w2_ref, b2_ref, perm):
    y = _conv3x3(xT, h, w, _reorder_w(w1_ref[0], perm), b1_ref[0])
    y = jax.nn.gelu(y)
    return _conv3x3(y, h, w, _reorder_w(w2_ref[0], perm), b2_ref[0])


def _upsample_mat(h, w):
    q = jax.lax.broadcasted_iota(jnp.int32, (h * w, 4 * h * w), 1)
    p = jax.lax.broadcasted_iota(jnp.int32, (h * w, 4 * h * w), 0)
    qh = q // (2 * w)
    qw = jax.lax.rem(q, 2 * w)
    src = (qh // 2) * w + qw // 2
    return (p == src).astype(jnp.float32)


def _pool_select_mat(h, w):
    q = jax.lax.broadcasted_iota(jnp.int32, (4 * h * w, h * w), 0)
    p = jax.lax.broadcasted_iota(jnp.int32, (4 * h * w, h * w), 1)
    ph = p // w
    pw = jax.lax.rem(p, w)
    src = (2 * ph) * (2 * w) + 2 * pw
    return (q == src).astype(jnp.float32)


def _pool_max2(y, w2):
    m1 = jnp.maximum(y, jnp.roll(y, -1, axis=1))
    return jnp.maximum(m1, jnp.roll(m1, -w2, axis=1))


def _mm(a, b):
    return jax.lax.dot_general(
        a, b, (((1,), (0,)), ((), ())), preferred_element_type=jnp.float32
    )


def _expert_kernel(
    idx_ref, x_ref, w1_ref, b1_ref, w2_ref, b2_ref, g_ref, p_ref, out_ref
):
    s = pl.program_id(0)
    e = idx_ref[s]
    cls = jax.lax.rem(e, 3)
    g = g_ref[s, 0]
    xT = x_ref[0]
    perm = p_ref[...]

    def _accum(y):
        contrib = g * y

        @pl.when(s % _K == 0)
        def _():
            out_ref[0] = contrib

        @pl.when(s % _K != 0)
        def _():
            out_ref[0] = out_ref[0] + contrib

    @pl.when(cls == 0)
    def _():
        _accum(_expert_block(xT, _H, _W, w1_ref, b1_ref, w2_ref, b2_ref, perm))

    @pl.when(cls == 1)
    def _():
        xup = _mm(xT, _upsample_mat(_H, _W))
        y = _expert_block(
            xup, 2 * _H, 2 * _W, w1_ref, b1_ref, w2_ref, b2_ref, perm
        )
        _accum(_mm(_pool_max2(y, 2 * _W), _pool_select_mat(_H, _W)))

    @pl.when(cls == 2)
    def _():
        xdn = _mm(_pool_max2(xT, _W), _pool_select_mat(_H // 2, _W // 2))
        y = _expert_block(
            xdn, _H // 2, _W // 2, w1_ref, b1_ref, w2_ref, b2_ref, perm
        )
        _accum(_mm(y, _upsample_mat(_H // 2, _W // 2)))


def kernel(x, text_feature, Wg_x, Wg_t, bg, W1, b1, W2, b2):
    x_chw = x.reshape(_B, _C, _HW)

    idx, topg, aux = pl.pallas_call(
        _gate_kernel,
        out_shape=[
            jax.ShapeDtypeStruct((_B, _K), jnp.int32),
            jax.ShapeDtypeStruct((_B, _K), jnp.float32),
            jax.ShapeDtypeStruct((1, 1), jnp.float32),
        ],
    )(x_chw, text_feature, Wg_x, Wg_t, bg.reshape(1, _E))

    idx_flat = idx.reshape(_B * _K)
    g_flat = topg.reshape(_B * _K, 1)

    c = jnp.arange(9 * _C)
    perm_mat = (
        jnp.arange(9 * _C)[:, None] == ((c % _C) * 9 + c // _C)[None, :]
    ).astype(jnp.bfloat16)

    grid_spec = pltpu.PrefetchScalarGridSpec(
        num_scalar_prefetch=1,
        grid=(_B * _K,),
        in_specs=[
            pl.BlockSpec((1, _C, _HW), lambda s, idx: (s // _K, 0, 0)),
            pl.BlockSpec((1, _C, 9 * _C), lambda s, idx: (idx[s], 0, 0)),
            pl.BlockSpec((1, _C, 1), lambda s, idx: (idx[s], 0, 0)),
            pl.BlockSpec((1, _C, 9 * _C), lambda s, idx: (idx[s], 0, 0)),
            pl.BlockSpec((1, _C, 1), lambda s, idx: (idx[s], 0, 0)),
            pl.BlockSpec((_B * _K, 1), lambda s, idx: (0, 0)),
            pl.BlockSpec((9 * _C, 9 * _C), lambda s, idx: (0, 0)),
        ],
        out_specs=pl.BlockSpec((1, _C, _HW), lambda s, idx: (s // _K, 0, 0)),
    )
    out_flat = pl.pallas_call(
        _expert_kernel,
        grid_spec=grid_spec,
        out_shape=jax.ShapeDtypeStruct((_B, _C, _HW), jnp.float32),
    )(
        idx_flat,
        x_chw,
        W1.reshape(_E, _C, _C * 9),
        b1.reshape(_E, _C, 1),
        W2.reshape(_E, _C, _C * 9),
        b2.reshape(_E, _C, 1),
        g_flat,
        perm_mat,
    )

    return out_flat.reshape(_B, _C, _H, _W), aux.reshape(())


# speedup vs baseline: 2.3275x; 1.0130x over previous
import numpy as np
import jax
import jax.numpy as jnp
from jax.experimental import pallas as pl
from jax.experimental.pallas import tpu as pltpu

_B, _C, _H, _W = 4, 192, 24, 24
_DT, _E, _K = 512, 8, 2
_HW = _H * _W


def _gate_kernel(x_ref, t_ref, wx_ref, wt_ref, bg_ref, idx_ref, g_ref, aux_ref):
    xp = jnp.mean(x_ref[...], axis=2)
    logits = (
        jnp.dot(xp, wx_ref[...], preferred_element_type=jnp.float32)
        + jnp.dot(t_ref[...], wt_ref[...], preferred_element_type=jnp.float32)
        + bg_ref[...]
    )
    ii = jax.lax.broadcasted_iota(jnp.int32, (_B, _E), 1)
    m1 = jnp.max(logits, axis=1, keepdims=True)
    i1 = jnp.min(jnp.where(logits == m1, ii, _E), axis=1, keepdims=True)
    masked = jnp.where(ii == i1, -jnp.inf, logits)
    m2 = jnp.max(masked, axis=1, keepdims=True)
    i2 = jnp.min(jnp.where(masked == m2, ii, _E), axis=1, keepdims=True)
    g1 = 1.0 / (1.0 + jnp.exp(m2 - m1))
    g2 = 1.0 - g1
    sel1 = ii == i1
    sel2 = ii == i2
    gates = jnp.where(sel1, g1, 0.0) + jnp.where(sel2, g2, 0.0)
    importance = jnp.sum(gates, axis=0, keepdims=True)
    load = jnp.sum((sel1 | sel2).astype(jnp.float32), axis=0, keepdims=True)

    def _cv(v):
        m = jnp.mean(v)
        var = jnp.mean(v * v) - m * m
        return var / (m * m + 1e-10)

    aux_ref[...] = (_cv(importance) + _cv(load)).reshape(1, 1)
    idx_ref[...] = jnp.concatenate([i1, i2], axis=1)
    g_ref[...] = jnp.concatenate([g1, g2], axis=1)


def _conv3x3(xb, h, w, wb, bvec):
    hw = h * w
    p = jax.lax.broadcasted_iota(jnp.int32, (1, hw), 1)
    ph = p // w
    pw = jax.lax.rem(p, w)
    cols = []
    for dh in (-1, 0, 1):
        for dw in (-1, 0, 1):
            s = dh * w + dw
            xs = xb if s == 0 else jnp.roll(xb, -s, axis=1)
            m = (ph + dh >= 0) & (ph + dh < h) & (pw + dw >= 0) & (pw + dw < w)
            cols.append(jnp.where(m, xs, jnp.bfloat16(0)))
    xcol = jnp.concatenate(cols, axis=0)
    y = jax.lax.dot_general(
        wb, xcol, (((1,), (0,)), ((), ())), preferred_element_type=jnp.float32
    )
    return y + bvec


def _expert_block(xb, h, w, w1b, b1_ref, w2b, b2_ref):
    y = _conv3x3(xb, h, w, w1b, b1_ref[0])
    y = jax.nn.gelu(y)
    return _conv3x3(y.astype(jnp.bfloat16), h, w, w2b, b2_ref[0])


def _pool_max2(y, w2):
    m1 = jnp.maximum(y, jnp.roll(y, -1, axis=1))
    return jnp.maximum(m1, jnp.roll(m1, -w2, axis=1))


def _mmb(a, b):
    return jax.lax.dot_general(
        a.astype(jnp.bfloat16),
        b,
        (((1,), (0,)), ((), ())),
        preferred_element_type=jnp.float32,
    )


def _expert_kernel(
    idx_ref,
    x_ref,
    w1_ref,
    b1_ref,
    w2_ref,
    b2_ref,
    g_ref,
    p_ref,
    up24_ref,
    sel24_ref,
    sel12_ref,
    up12_ref,
    out_ref,
):
    s = pl.program_id(0)
    e = idx_ref[s]
    cls = jax.lax.rem(e, 3)
    g = g_ref[s, 0]
    xT = x_ref[0]

    def _reorder(wn_ref):
        return jax.lax.dot_general(
            wn_ref[0].astype(jnp.bfloat16),
            p_ref[...],
            (((1,), (0,)), ((), ())),
            preferred_element_type=jnp.float32,
        ).astype(jnp.bfloat16)

    w1b = _reorder(w1_ref)
    w2b = _reorder(w2_ref)

    def _accum(y):
        contrib = g * y

        @pl.when(s % _K == 0)
        def _():
            out_ref[0] = contrib

        @pl.when(s % _K != 0)
        def _():
            out_ref[0] = out_ref[0] + contrib

    @pl.when(cls == 0)
    def _():
        y = _expert_block(
            xT.astype(jnp.bfloat16), _H, _W, w1b, b1_ref, w2b, b2_ref
        )
        _accum(y)

    @pl.when(cls == 1)
    def _():
        xup = _mmb(xT, up24_ref[...])
        y = _expert_block(
            xup.astype(jnp.bfloat16), 2 * _H, 2 * _W, w1b, b1_ref, w2b, b2_ref
        )
        _accum(_mmb(_pool_max2(y, 2 * _W), sel24_ref[...]))

    @pl.when(cls == 2)
    def _():
        xdn = _mmb(_pool_max2(xT, _W), sel12_ref[...])
        y = _expert_block(
            xdn.astype(jnp.bfloat16), _H // 2, _W // 2, w1b, b1_ref, w2b, b2_ref
        )
        _accum(_mmb(y, up12_ref[...]))


def _np_upsample_mat(h, w):
    q = np.arange(4 * h * w)
    src = (q // (2 * w) // 2) * w + (q % (2 * w)) // 2
    return (np.arange(h * w)[:, None] == src[None, :]).astype(np.float32)


def _np_pool_select_mat(h, w):
    p = np.arange(h * w)
    src = (2 * (p // w)) * (2 * w) + 2 * (p % w)
    return (np.arange(4 * h * w)[:, None] == src[None, :]).astype(np.float32)


def _np_perm_mat():
    c = np.arange(9 * _C)
    src = (c % _C) * 9 + c // _C
    return (np.arange(9 * _C)[:, None] == src[None, :]).astype(np.float32)


_PERM = jnp.asarray(_np_perm_mat(), dtype=jnp.bfloat16)
_UP24 = jnp.asarray(_np_upsample_mat(_H, _W), dtype=jnp.bfloat16)
_SEL24 = jnp.asarray(_np_pool_select_mat(_H, _W), dtype=jnp.bfloat16)
_SEL12 = jnp.asarray(_np_pool_select_mat(_H // 2, _W // 2), dtype=jnp.bfloat16)
_UP12 = jnp.asarray(_np_upsample_mat(_H // 2, _W // 2), dtype=jnp.bfloat16)


def kernel(x, text_feature, Wg_x, Wg_t, bg, W1, b1, W2, b2):
    x_chw = x.reshape(_B, _C, _HW)

    idx, topg, aux = pl.pallas_call(
        _gate_kernel,
        out_shape=[
            jax.ShapeDtypeStruct((_B, _K), jnp.int32),
            jax.ShapeDtypeStruct((_B, _K), jnp.float32),
            jax.ShapeDtypeStruct((1, 1), jnp.float32),
        ],
    )(x_chw, text_feature, Wg_x, Wg_t, bg.reshape(1, _E))

    idx_flat = idx.reshape(_B * _K)
    g_flat = topg.reshape(_B * _K, 1)

    def _const_spec(shape):
        return pl.BlockSpec(shape, lambda s, idx: tuple(0 for _ in shape))

    grid_spec = pltpu.PrefetchScalarGridSpec(
        num_scalar_prefetch=1,
        grid=(_B * _K,),
        in_specs=[
            pl.BlockSpec((1, _C, _HW), lambda s, idx: (s // _K, 0, 0)),
            pl.BlockSpec((1, _C, 9 * _C), lambda s, idx: (idx[s], 0, 0)),
            pl.BlockSpec((1, _C, 1), lambda s, idx: (idx[s], 0, 0)),
            pl.BlockSpec((1, _C, 9 * _C), lambda s, idx: (idx[s], 0, 0)),
            pl.BlockSpec((1, _C, 1), lambda s, idx: (idx[s], 0, 0)),
            _const_spec((_B * _K, 1)),
            _const_spec((9 * _C, 9 * _C)),
            _const_spec((_HW, 4 * _HW)),
            _const_spec((4 * _HW, _HW)),
            _const_spec((_HW, _HW // 4)),
            _const_spec((_HW // 4, _HW)),
        ],
        out_specs=pl.BlockSpec((1, _C, _HW), lambda s, idx: (s // _K, 0, 0)),
    )
    out_flat = pl.pallas_call(
        _expert_kernel,
        grid_spec=grid_spec,
        out_shape=jax.ShapeDtypeStruct((_B, _C, _HW), jnp.float32),
    )(
        idx_flat,
        x_chw,
        W1.reshape(_E, _C, _C * 9),
        b1.reshape(_E, _C, 1),
        W2.reshape(_E, _C, _C * 9),
        b2.reshape(_E, _C, 1),
        g_flat,
        _PERM,
        _UP24,
        _SEL24,
        _SEL12,
        _UP12,
    )

    return out_flat.reshape(_B, _C, _H, _W), aux.reshape(())
